# baseline (speedup 1.0000x reference)
import functools

import jax
import jax.numpy as jnp
from jax import lax
from jax.experimental import pallas as pl
from jax.experimental.pallas import tpu as pltpu
from jax.experimental.pallas import tpu_sc as plsc

D = 32
MAXLEN = 200
NC = 2
NS = 16
NW = NC * NS
K = 80
NSTR = 10
CH = K * NSTR
SEQS_PER_CHUNK = CH // MAXLEN


@functools.lru_cache(maxsize=None)
def _build(n_rows):
    rows_per_worker = n_rows // NW
    n_chunks = rows_per_worker // CH
    mesh = plsc.VectorSubcoreMesh(core_axis_name="c", subcore_axis_name="s")

    @functools.partial(
        pl.kernel,
        mesh=mesh,
        out_type=jax.ShapeDtypeStruct((n_rows, D), jnp.float32),
        scratch_types=[
            pltpu.VMEM((CH,), jnp.int32),
            pltpu.VMEM((CH,), jnp.int32),
            pltpu.VMEM((CH, D), jnp.float32),
            pltpu.VMEM((CH, D), jnp.float32),
            pltpu.VMEM((MAXLEN, D), jnp.float32),
            pltpu.SemaphoreType.DMA,
            pltpu.SemaphoreType.DMA,
            pltpu.SemaphoreType.DMA,
            pltpu.SemaphoreType.DMA,
            pltpu.SemaphoreType.DMA,
            pltpu.SemaphoreType.DMA,
        ],
        compiler_params=pltpu.CompilerParams(use_tc_tiling_on_sc=False),
    )
    def emb(ids_hbm, tok_hbm, pos_hbm, out_hbm,
            idx0, idx1, rows0, rows1, pos_v,
            gsem0, gsem1, isem0, isem1, osem0, osem1):
        idx = (idx0, idx1)
        rows = (rows0, rows1)
        gsem = (gsem0, gsem1)
        isem = (isem0, isem1)
        osem = (osem0, osem1)

        wid = lax.axis_index("s") * NC + lax.axis_index("c")
        base = wid * rows_per_worker
        pltpu.sync_copy(pos_hbm, pos_v)

        def fire_gathers(b):
            def fire_body(j16, c):
                v = idx[b][pl.ds(j16 * 16, 16)]
                for u in range(16):
                    pltpu.async_copy(tok_hbm.at[pl.ds(v[u], 1)],
                                     rows[b].at[pl.ds(j16 * 16 + u, 1)],
                                     gsem[b])
                return c
            lax.fori_loop(0, CH // 16, fire_body, 0)

        def drain_gathers(b):
            pltpu.make_async_copy(out_hbm.at[pl.ds(0, CH)], rows[b],
                                  gsem[b]).wait()

        def drain_idx(b):
            pltpu.make_async_copy(ids_hbm.at[pl.ds(0, CH)], idx[b],
                                  isem[b]).wait()

        def drain_out(b):
            pltpu.make_async_copy(rows[b], out_hbm.at[pl.ds(0, CH)],
                                  osem[b]).wait()

        def add_pos(b):
            def add_body(r2, c2):
                for u in range(8):
                    r = r2 * 8 + u
                    p0 = pos_v[r, pl.ds(0, 16)]
                    p1 = pos_v[r, pl.ds(16, 16)]
                    for s in range(SEQS_PER_CHUNK):
                        row = s * MAXLEN + r
                        plsc.addupdate(rows[b].at[row, pl.ds(0, 16)], p0)
                        plsc.addupdate(rows[b].at[row, pl.ds(16, 16)], p1)
                return c2
            lax.fori_loop(0, MAXLEN // 8, add_body, 0)

        pltpu.sync_copy(ids_hbm.at[pl.ds(base, CH)], idx0)
        fire_gathers(0)
        pltpu.async_copy(ids_hbm.at[pl.ds(base + CH, CH)], idx1, isem1)

        def pair_body(go, carry):
            for par in range(2):
                b, nb = par, 1 - par
                g = 2 * go + par
                start = base + g * CH

                @pl.when(jnp.logical_and(g >= 1, g + 1 < n_chunks))
                def _wait_prev_out():
                    drain_out(nb)

                @pl.when(g + 1 < n_chunks)
                def _fire_next():
                    drain_idx(nb)
                    fire_gathers(nb)

                drain_gathers(b)

                @pl.when(g + 2 < n_chunks)
                def _prefetch_idx():
                    pltpu.async_copy(
                        ids_hbm.at[pl.ds(start + 2 * CH, CH)], idx[b],
                        isem[b])

                add_pos(b)
                pltpu.async_copy(rows[b], out_hbm.at[pl.ds(start, CH)],
                                 osem[b])
            return carry

        lax.fori_loop(0, n_chunks // 2, pair_body, 0)
        drain_out(0)
        drain_out(1)

    return emb


def kernel(token_ids, token_table, pos_table):
    batch, seq = token_ids.shape
    n_rows = batch * seq
    ids_flat = token_ids.astype(jnp.int32).reshape(n_rows)
    out = _build(n_rows)(ids_flat, token_table, pos_table)
    return out.reshape(batch, seq, D)


# speedup vs baseline: 1.0436x; 1.0436x over previous
import functools

import jax
import jax.numpy as jnp
from jax import lax
from jax.experimental import pallas as pl
from jax.experimental.pallas import tpu as pltpu
from jax.experimental.pallas import tpu_sc as plsc

D = 32
MAXLEN = 200
NC = 2
NS = 16
NW = NC * NS
K = 80
NSTR = 10
CH = K * NSTR
SEQS_PER_CHUNK = CH // MAXLEN


@functools.lru_cache(maxsize=None)
def _build(n_rows):
    rows_per_worker = n_rows // NW
    n_chunks = rows_per_worker // CH
    mesh = plsc.VectorSubcoreMesh(core_axis_name="c", subcore_axis_name="s")

    @functools.partial(
        pl.kernel,
        mesh=mesh,
        out_type=jax.ShapeDtypeStruct((n_rows, D), jnp.float32),
        scratch_types=[
            pltpu.VMEM((CH,), jnp.int32),
            pltpu.VMEM((CH,), jnp.int32),
            pltpu.VMEM((CH, D), jnp.float32),
            pltpu.VMEM((CH, D), jnp.float32),
            pltpu.VMEM((MAXLEN, D), jnp.float32),
            pltpu.SemaphoreType.DMA,
            pltpu.SemaphoreType.DMA,
            pltpu.SemaphoreType.DMA,
            pltpu.SemaphoreType.DMA,
            pltpu.SemaphoreType.DMA,
            pltpu.SemaphoreType.DMA,
        ],
        compiler_params=pltpu.CompilerParams(use_tc_tiling_on_sc=False),
    )
    def emb(ids_hbm, tok_hbm, pos_hbm, out_hbm,
            idx0, idx1, rows0, rows1, pos_v,
            gsem0, gsem1, isem0, isem1, osem0, osem1):
        idx = (idx0, idx1)
        rows = (rows0, rows1)
        gsem = (gsem0, gsem1)
        isem = (isem0, isem1)
        osem = (osem0, osem1)

        wid = lax.axis_index("s") * NC + lax.axis_index("c")
        base = wid * rows_per_worker
        pltpu.sync_copy(pos_hbm, pos_v)

        def fire_gathers(b):
            for j in range(NSTR):
                pltpu.async_copy(tok_hbm.at[idx[b].at[pl.ds(j * K, K)]],
                                 rows[b].at[pl.ds(j * K, K)], gsem[b])

        def drain_gathers(b):
            pltpu.make_async_copy(out_hbm.at[pl.ds(0, CH)], rows[b],
                                  gsem[b]).wait()

        def drain_idx(b):
            pltpu.make_async_copy(ids_hbm.at[pl.ds(0, CH)], idx[b],
                                  isem[b]).wait()

        def drain_out(b):
            pltpu.make_async_copy(rows[b], out_hbm.at[pl.ds(0, CH)],
                                  osem[b]).wait()

        def add_pos(b):
            def add_body(r2, c2):
                for u in range(8):
                    r = r2 * 8 + u
                    p0 = pos_v[r, pl.ds(0, 16)]
                    p1 = pos_v[r, pl.ds(16, 16)]
                    for s in range(SEQS_PER_CHUNK):
                        row = s * MAXLEN + r
                        plsc.addupdate(rows[b].at[row, pl.ds(0, 16)], p0)
                        plsc.addupdate(rows[b].at[row, pl.ds(16, 16)], p1)
                return c2
            lax.fori_loop(0, MAXLEN // 8, add_body, 0)

        pltpu.sync_copy(ids_hbm.at[pl.ds(base, CH)], idx0)
        fire_gathers(0)
        pltpu.async_copy(ids_hbm.at[pl.ds(base + CH, CH)], idx1, isem1)

        def pair_body(go, carry):
            for par in range(2):
                b, nb = par, 1 - par
                g = 2 * go + par
                start = base + g * CH

                @pl.when(jnp.logical_and(g >= 1, g + 1 < n_chunks))
                def _wait_prev_out():
                    drain_out(nb)

                @pl.when(g + 1 < n_chunks)
                def _fire_next():
                    drain_idx(nb)
                    fire_gathers(nb)

                drain_gathers(b)

                @pl.when(g + 2 < n_chunks)
                def _prefetch_idx():
                    pltpu.async_copy(
                        ids_hbm.at[pl.ds(start + 2 * CH, CH)], idx[b],
                        isem[b])

                add_pos(b)
                pltpu.async_copy(rows[b], out_hbm.at[pl.ds(start, CH)],
                                 osem[b])
            return carry

        lax.fori_loop(0, n_chunks // 2, pair_body, 0)
        drain_out(0)
        drain_out(1)

    return emb


def kernel(token_ids, token_table, pos_table):
    batch, seq = token_ids.shape
    n_rows = batch * seq
    ids_flat = token_ids.astype(jnp.int32).reshape(n_rows)
    out = _build(n_rows)(ids_flat, token_table, pos_table)
    return out.reshape(batch, seq, D)


# speedup vs baseline: 1.0477x; 1.0040x over previous
import functools

import jax
import jax.numpy as jnp
from jax import lax
from jax.experimental import pallas as pl
from jax.experimental.pallas import tpu as pltpu
from jax.experimental.pallas import tpu_sc as plsc

D = 32
MAXLEN = 200
NC = 2
NS = 16
NW = NC * NS
K = 80
NSTR = 20
CH = K * NSTR
SEQS_PER_CHUNK = CH // MAXLEN


@functools.lru_cache(maxsize=None)
def _build(n_rows):
    rows_per_worker = n_rows // NW
    n_chunks = rows_per_worker // CH
    mesh = plsc.VectorSubcoreMesh(core_axis_name="c", subcore_axis_name="s")

    @functools.partial(
        pl.kernel,
        mesh=mesh,
        out_type=jax.ShapeDtypeStruct((n_rows, D), jnp.float32),
        scratch_types=[
            pltpu.VMEM((CH,), jnp.int32),
            pltpu.VMEM((CH,), jnp.int32),
            pltpu.VMEM((CH, D), jnp.float32),
            pltpu.VMEM((CH, D), jnp.float32),
            pltpu.VMEM((MAXLEN, D), jnp.float32),
            pltpu.SemaphoreType.DMA,
            pltpu.SemaphoreType.DMA,
            pltpu.SemaphoreType.DMA,
            pltpu.SemaphoreType.DMA,
            pltpu.SemaphoreType.DMA,
            pltpu.SemaphoreType.DMA,
        ],
        compiler_params=pltpu.CompilerParams(use_tc_tiling_on_sc=False),
    )
    def emb(ids_hbm, tok_hbm, pos_hbm, out_hbm,
            idx0, idx1, rows0, rows1, pos_v,
            gsem0, gsem1, isem0, isem1, osem0, osem1):
        idx = (idx0, idx1)
        rows = (rows0, rows1)
        gsem = (gsem0, gsem1)
        isem = (isem0, isem1)
        osem = (osem0, osem1)

        wid = lax.axis_index("s") * NC + lax.axis_index("c")
        base = wid * rows_per_worker
        pltpu.sync_copy(pos_hbm, pos_v)

        def fire_gathers(b):
            for j in range(NSTR):
                pltpu.async_copy(tok_hbm.at[idx[b].at[pl.ds(j * K, K)]],
                                 rows[b].at[pl.ds(j * K, K)], gsem[b])

        def drain_gathers(b):
            pltpu.make_async_copy(out_hbm.at[pl.ds(0, CH)], rows[b],
                                  gsem[b]).wait()

        def drain_idx(b):
            pltpu.make_async_copy(ids_hbm.at[pl.ds(0, CH)], idx[b],
                                  isem[b]).wait()

        def drain_out(b):
            pltpu.make_async_copy(rows[b], out_hbm.at[pl.ds(0, CH)],
                                  osem[b]).wait()

        def add_pos(b):
            def add_body(r2, c2):
                for u in range(8):
                    r = r2 * 8 + u
                    p0 = pos_v[r, pl.ds(0, 16)]
                    p1 = pos_v[r, pl.ds(16, 16)]
                    for s in range(SEQS_PER_CHUNK):
                        row = s * MAXLEN + r
                        plsc.addupdate(rows[b].at[row, pl.ds(0, 16)], p0)
                        plsc.addupdate(rows[b].at[row, pl.ds(16, 16)], p1)
                return c2
            lax.fori_loop(0, MAXLEN // 8, add_body, 0)

        pltpu.sync_copy(ids_hbm.at[pl.ds(base, CH)], idx0)
        fire_gathers(0)
        pltpu.async_copy(ids_hbm.at[pl.ds(base + CH, CH)], idx1, isem1)

        def pair_body(go, carry):
            for par in range(2):
                b, nb = par, 1 - par
                g = 2 * go + par
                start = base + g * CH

                @pl.when(jnp.logical_and(g >= 1, g + 1 < n_chunks))
                def _wait_prev_out():
                    drain_out(nb)

                @pl.when(g + 1 < n_chunks)
                def _fire_next():
                    drain_idx(nb)
                    fire_gathers(nb)

                drain_gathers(b)

                @pl.when(g + 2 < n_chunks)
                def _prefetch_idx():
                    pltpu.async_copy(
                        ids_hbm.at[pl.ds(start + 2 * CH, CH)], idx[b],
                        isem[b])

                add_pos(b)
                pltpu.async_copy(rows[b], out_hbm.at[pl.ds(start, CH)],
                                 osem[b])
            return carry

        lax.fori_loop(0, n_chunks // 2, pair_body, 0)
        drain_out(0)
        drain_out(1)

    return emb


def kernel(token_ids, token_table, pos_table):
    batch, seq = token_ids.shape
    n_rows = batch * seq
    ids_flat = token_ids.astype(jnp.int32).reshape(n_rows)
    out = _build(n_rows)(ids_flat, token_table, pos_table)
    return out.reshape(batch, seq, D)
